# baseline (speedup 1.0000x reference)
import functools

import jax
import jax.numpy as jnp
from jax import lax
from jax.experimental import pallas as pl
from jax.experimental.pallas import tpu as pltpu
from jax.experimental.pallas import tpu_sc as plsc

B = 4096
S = 200
V = 1000
VP = 1024
QTR = VP // 4
E = 128
OUT = 128

NC, NS, L = 2, 16, 16
NW = NC * NS
ROWS_PER_W = B // NW
SUB = 32
N_SUB = ROWS_PER_W // SUB
N_FULL = S // L
REM = S - N_FULL * L


def _sc_hist_body(x_hbm, h_hbm, x_v, h0_v, h1_v, sem_x, sem_h0, sem_h1):
    wid = lax.axis_index("s") * NC + lax.axis_index("c")
    row0 = wid * ROWS_PER_W
    cp_x = pltpu.async_copy(x_hbm.at[pl.ds(row0, ROWS_PER_W)], x_v, sem_x)
    zeros16 = jnp.zeros((L,), jnp.int32)
    rem_mask = lax.iota(jnp.int32, L) >= (L - REM)
    hbufs = (h0_v, h1_v)
    hsems = (sem_h0, sem_h1)

    def zero(buf):
        def zero_body(r, _):
            for k in range(QTR // L):
                buf[r, pl.ds(k * L, L)] = zeros16
            return 0
        lax.fori_loop(0, SUB, zero_body, 0)

    def load_idx(row):
        idxs = [x_v[row, pl.ds(c * L, L)] for c in range(N_FULL)]
        idxs.append(x_v[row, pl.ds(S - L, L)])
        return idxs

    zero(h0_v)
    cp_x.wait()

    out_cps = [None, None]
    for s in range(N_SUB):
        buf, sem = hbufs[s % 2], hsems[s % 2]
        if s >= 2:
            out_cps[s % 2].wait()
            zero(buf)

        def row_body(r2, _):
            for half in range(2):
                r = r2 * 2 + half
                rvec = jnp.full((L,), r, jnp.int32)
                idxs = load_idx(s * SUB + r)
                words = [idx & (QTR - 1) for idx in idxs]
                vals = [
                    lax.shift_left(
                        jnp.full((L,), 1, jnp.int32),
                        lax.shift_left(lax.shift_right_logical(idx, 8), 3),
                    )
                    for idx in idxs
                ]
                for c in range(N_FULL):
                    plsc.addupdate_scatter(buf, [rvec, words[c]], vals[c])
                plsc.addupdate_scatter(buf, [rvec, words[N_FULL]],
                                       vals[N_FULL], mask=rem_mask)
            return 0
        lax.fori_loop(0, SUB // 2, row_body, 0)

        out_cps[s % 2] = pltpu.async_copy(
            buf, h_hbm.at[pl.ds(row0 + s * SUB, SUB)], sem
        )
        if s == 0:
            zero(h1_v)
    out_cps[0].wait()
    out_cps[1].wait()


def _sc_hist(x):
    mesh = plsc.VectorSubcoreMesh(
        core_axis_name="c", subcore_axis_name="s", num_cores=NC, num_subcores=NS
    )
    f = pl.kernel(
        _sc_hist_body,
        out_type=jax.ShapeDtypeStruct((B, QTR), jnp.int32),
        mesh=mesh,
        scratch_types=[
            pltpu.VMEM((ROWS_PER_W, S), jnp.int32),
            pltpu.VMEM((SUB, QTR), jnp.int32),
            pltpu.VMEM((SUB, QTR), jnp.int32),
            pltpu.SemaphoreType.DMA,
            pltpu.SemaphoreType.DMA,
            pltpu.SemaphoreType.DMA,
        ],
        compiler_params=pltpu.CompilerParams(needs_layout_passes=False),
        name="sc_row_histogram",
    )
    return f(x)


def _tc_fold_body(tbl_ref, w_ref, m_ref):
    m = lax.dot_general(
        tbl_ref[...], w_ref[...], (((1,), (1,)), ((), ())),
        preferred_element_type=jnp.float32,
    )
    m_ref[...] = (m * (1.0 / S)).astype(jnp.bfloat16)


def _tc_fold(tbl_p, W):
    return pl.pallas_call(
        _tc_fold_body,
        out_shape=jax.ShapeDtypeStruct((VP, E), jnp.bfloat16),
    )(tbl_p, W)


def _tc_mm_body(h_ref, m_ref, b_ref, out_ref):
    h32 = h_ref[...]
    acc = b_ref[...] + jnp.zeros(out_ref.shape, jnp.float32)
    for q in range(4):
        byte = lax.shift_right_logical(h32, 8 * q) & 0xFF
        cnt = byte.astype(jnp.float32).astype(jnp.bfloat16)
        acc += lax.dot_general(
            cnt, m_ref[q * QTR:(q + 1) * QTR, :], (((1,), (0,)), ((), ())),
            preferred_element_type=jnp.float32,
        )
    out_ref[...] = acc


def _tc_mm(H, M, b2):
    BB = 4096
    return pl.pallas_call(
        _tc_mm_body,
        grid=(B // BB,),
        in_specs=[
            pl.BlockSpec((BB, QTR), lambda i: (i, 0)),
            pl.BlockSpec((VP, E), lambda i: (0, 0)),
            pl.BlockSpec((1, OUT), lambda i: (0, 0)),
        ],
        out_specs=pl.BlockSpec((BB, OUT), lambda i: (i, 0)),
        out_shape=jax.ShapeDtypeStruct((B, OUT), jnp.float32),
    )(H, M, b2)


def kernel(x, table, W, b):
    H = _sc_hist(x)
    tbl_p = (
        jnp.zeros((VP, E), jnp.float32).at[:V].set(table).at[0].set(0.0)
    )
    M = _tc_fold(tbl_p, W)
    return _tc_mm(H, M, b.reshape(1, OUT))


# speedup vs baseline: 1.0161x; 1.0161x over previous
import jax
import jax.numpy as jnp
from jax import lax
from jax.experimental import pallas as pl
from jax.experimental.pallas import tpu as pltpu
from jax.experimental.pallas import tpu_sc as plsc

B = 4096
S = 200
V = 1000
VP = 1024
QTR = VP // 4
E = 128
OUT = 128

NC, NS, L = 2, 16, 16
NW = NC * NS
ROWS_PER_W = B // NW
SUB = 32
N_SUB = ROWS_PER_W // SUB
N_FULL = S // L
REM = S - N_FULL * L


def _sc_hist_body(x_hbm, h_hbm, x_v, h0_v, h1_v, sem_x, sem_h0, sem_h1):
    wid = lax.axis_index("s") * NC + lax.axis_index("c")
    row0 = wid * ROWS_PER_W
    cp_x = pltpu.async_copy(x_hbm.at[pl.ds(row0, ROWS_PER_W)], x_v, sem_x)
    zeros16 = jnp.zeros((L,), jnp.int32)
    rem_mask = lax.iota(jnp.int32, L) >= (L - REM)
    hbufs = (h0_v, h1_v)
    hsems = (sem_h0, sem_h1)

    def zero(buf):
        def zero_body(r, _):
            for k in range(QTR // L):
                buf[r, pl.ds(k * L, L)] = zeros16
            return 0
        lax.fori_loop(0, SUB, zero_body, 0)

    def load_idx(row):
        idxs = [x_v[row, pl.ds(c * L, L)] for c in range(N_FULL)]
        idxs.append(x_v[row, pl.ds(S - L, L)])
        return idxs

    zero(h0_v)
    cp_x.wait()

    out_cps = [None, None]
    for s in range(N_SUB):
        buf, sem = hbufs[s % 2], hsems[s % 2]
        if s >= 2:
            out_cps[s % 2].wait()
            zero(buf)

        def row_body(r2, _):
            for half in range(2):
                r = r2 * 2 + half
                rvec = jnp.full((L,), r, jnp.int32)
                idxs = load_idx(s * SUB + r)
                words = [idx & (QTR - 1) for idx in idxs]
                vals = [
                    lax.shift_left(
                        jnp.full((L,), 1, jnp.int32),
                        lax.shift_left(lax.shift_right_logical(idx, 8), 3),
                    )
                    for idx in idxs
                ]
                for c in range(N_FULL):
                    plsc.addupdate_scatter(buf, [rvec, words[c]], vals[c])
                plsc.addupdate_scatter(buf, [rvec, words[N_FULL]],
                                       vals[N_FULL], mask=rem_mask)
            return 0
        lax.fori_loop(0, SUB // 2, row_body, 0)

        out_cps[s % 2] = pltpu.async_copy(
            buf, h_hbm.at[pl.ds(row0 + s * SUB, SUB)], sem
        )
        if s == 0:
            zero(h1_v)
    out_cps[0].wait()
    out_cps[1].wait()


def _sc_hist(x):
    mesh = plsc.VectorSubcoreMesh(
        core_axis_name="c", subcore_axis_name="s", num_cores=NC, num_subcores=NS
    )
    f = pl.kernel(
        _sc_hist_body,
        out_type=jax.ShapeDtypeStruct((B, QTR), jnp.int32),
        mesh=mesh,
        scratch_types=[
            pltpu.VMEM((ROWS_PER_W, S), jnp.int32),
            pltpu.VMEM((SUB, QTR), jnp.int32),
            pltpu.VMEM((SUB, QTR), jnp.int32),
            pltpu.SemaphoreType.DMA,
            pltpu.SemaphoreType.DMA,
            pltpu.SemaphoreType.DMA,
        ],
        compiler_params=pltpu.CompilerParams(needs_layout_passes=False),
        name="sc_row_histogram",
    )
    return f(x)


def _tc_fold_body(tbl_ref, w_ref, m_ref):
    m = lax.dot_general(
        tbl_ref[...], w_ref[...], (((1,), (1,)), ((), ())),
        preferred_element_type=jnp.float32,
    )
    m_ref[...] = (m * (1.0 / S)).astype(jnp.bfloat16)


def _tc_fold(tbl_p, W):
    return pl.pallas_call(
        _tc_fold_body,
        out_shape=jax.ShapeDtypeStruct((VP, E), jnp.bfloat16),
    )(tbl_p, W)


def _tc_mm_body(h_ref, m_ref, b_ref, out_ref):
    h32 = h_ref[...]
    acc = b_ref[...] + jnp.zeros(out_ref.shape, jnp.float32)
    for q in range(4):
        byte = lax.shift_right_logical(h32, 8 * q) & 0xFF
        cnt = byte.astype(jnp.float32).astype(jnp.bfloat16)
        acc += lax.dot_general(
            cnt, m_ref[q * QTR:(q + 1) * QTR, :], (((1,), (0,)), ((), ())),
            preferred_element_type=jnp.float32,
        )
    out_ref[...] = acc


def _tc_mm(H, M, b2):
    BB = 2048
    return pl.pallas_call(
        _tc_mm_body,
        grid=(B // BB,),
        in_specs=[
            pl.BlockSpec((BB, QTR), lambda i: (i, 0)),
            pl.BlockSpec((VP, E), lambda i: (0, 0)),
            pl.BlockSpec((1, OUT), lambda i: (0, 0)),
        ],
        out_specs=pl.BlockSpec((BB, OUT), lambda i: (i, 0)),
        out_shape=jax.ShapeDtypeStruct((B, OUT), jnp.float32),
    )(H, M, b2)


def kernel(x, table, W, b):
    H = _sc_hist(x)
    tbl_p = (
        jnp.zeros((VP, E), jnp.float32).at[:V].set(table).at[0].set(0.0)
    )
    M = _tc_fold(tbl_p, W)
    return _tc_mm(H, M, b.reshape(1, OUT))
